# Initial kernel scaffold; baseline (speedup 1.0000x reference)
import jax
import jax.numpy as jnp
from jax.experimental import pallas as pl


def kernel(x, edge_index, edge_attr, W, b, root_emb, bond_emb0, bond_emb1, bond_emb2):
    raise NotImplementedError("write your pallas kernel here")


# speedup vs baseline: 9.2394x; 9.2394x over previous
import functools

import jax
import jax.numpy as jnp
from jax import lax
from jax.experimental import pallas as pl
from jax.experimental.pallas import tpu as pltpu
from jax.experimental.pallas import tpu_sc as plsc

N = 10000
E = 320000
D = 128
NT = 16
NC = 2
NPAD = 10240
SL = NPAD // NT

CH = 80
PER_TILE = (E // NC) // NT
CHUNKS = PER_TILE // CH
GROUPS = CH // 16


def _mm_body(x_ref, w_ref, b_ref, out_ref):
    y = lax.dot_general(x_ref[...], w_ref[...], (((1,), (1,)), ((), ())),
                        preferred_element_type=jnp.float32)
    out_ref[...] = y + b_ref[...]


def _matmul(xp, W, b):
    return pl.pallas_call(
        _mm_body,
        grid=(16,),
        in_specs=[
            pl.BlockSpec((SL, D), lambda i: (i, 0)),
            pl.BlockSpec((D, D), lambda i: (0, 0)),
            pl.BlockSpec((1, D), lambda i: (0, 0)),
        ],
        out_specs=pl.BlockSpec((SL, D), lambda i: (i, 0)),
        out_shape=jax.ShapeDtypeStruct((NPAD, D), jnp.float32),
    )(xp, W, b.reshape(1, D))


def _ec_body(a0, a1, a2, o):
    o[...] = a0[...] * 12 + a1[...] * 2 + a2[...]


def _edge_codes(a0, a1, a2):
    return pl.pallas_call(
        _ec_body,
        out_shape=jax.ShapeDtypeStruct((E,), jnp.int32),
    )(a0, a1, a2)


def _ct_body(b0, b1, b2, o):
    t = (b0[...][:, None, None, :] + b1[...][None, :, None, :]
         + b2[...][None, None, :, :])
    o[...] = t.reshape(60, D)


def _bond_table(b0, b1, b2):
    return pl.pallas_call(
        _ct_body,
        out_shape=jax.ShapeDtypeStruct((60, D), jnp.float32),
    )(b0, b1, b2)


def _deg_body(row_hbm, degp_hbm, deg_sp, rowbuf, ones_v, zbuf):
    c = lax.axis_index("c")
    s = lax.axis_index("s")

    def zero_blk(i, _):
        zbuf[pl.ds(i * 16, 16)] = jnp.zeros((16,), jnp.float32)
        return 0
    lax.fori_loop(0, SL // 16, zero_blk, 0)
    pltpu.sync_copy(zbuf, deg_sp.at[pl.ds(s * SL, SL)])

    def one_blk(i, _):
        ones_v[pl.ds(i * 16, 16)] = jnp.ones((16,), jnp.float32)
        return 0
    lax.fori_loop(0, GROUPS, one_blk, 0)
    plsc.subcore_barrier()

    base0 = c * (E // NC) + s * PER_TILE

    def chunk(j, _):
        pltpu.sync_copy(row_hbm.at[pl.ds(base0 + j * CH, CH)], rowbuf)
        pltpu.sync_copy(ones_v, deg_sp.at[rowbuf], add=True)
        return 0
    lax.fori_loop(0, CHUNKS, chunk, 0)
    plsc.subcore_barrier()

    @pl.when(s == 0)
    def _():
        pltpu.sync_copy(deg_sp, degp_hbm.at[c])


def _degree(row):
    mesh = plsc.VectorSubcoreMesh(core_axis_name="c", subcore_axis_name="s")
    f = functools.partial(
        pl.kernel,
        mesh=mesh,
        out_type=jax.ShapeDtypeStruct((NC, NPAD), jnp.float32),
        scratch_types=[
            pltpu.VMEM_SHARED((NPAD,), jnp.float32),
            pltpu.VMEM((CH,), jnp.int32),
            pltpu.VMEM((CH,), jnp.float32),
            pltpu.VMEM((SL,), jnp.float32),
        ],
    )(_deg_body)
    return f(row)


def _p3_body(h, degp, root, dis_o, self_o):
    deg = degp[0] + degp[1] + 1.0
    dis_o[...] = lax.rsqrt(deg)
    self_o[...] = jnp.maximum(h[...] + root[...], 0.0) * (1.0 / deg)[:, None]


def _norms(h, degp, root):
    return pl.pallas_call(
        _p3_body,
        out_shape=[
            jax.ShapeDtypeStruct((NPAD,), jnp.float32),
            jax.ShapeDtypeStruct((NPAD, D), jnp.float32),
        ],
    )(h, degp, root)


def _k3_body(h_hbm, dis_hbm, row_hbm, col_hbm, ec_hbm, ct_hbm, aggp_hbm,
             agg_sp, ct_v, dis_v, rowbuf, colbuf, ecbuf, bufG, bufM):
    c = lax.axis_index("c")
    s = lax.axis_index("s")
    r0 = s * SL

    def zrow(i, _):
        for q in range(D // 16):
            bufM[i, pl.ds(q * 16, 16)] = jnp.zeros((16,), jnp.float32)
        return 0
    lax.fori_loop(0, CH, zrow, 0)

    def zcp(i, _):
        pltpu.sync_copy(bufM, agg_sp.at[pl.ds(r0 + i * CH, CH)])
        return 0
    lax.fori_loop(0, SL // CH, zcp, 0)

    pltpu.sync_copy(ct_hbm, ct_v)
    pltpu.sync_copy(dis_hbm, dis_v.at[pl.ds(0, NPAD)])
    plsc.subcore_barrier()

    ebase = c * (E // NC) + s * PER_TILE

    def chunk(j, _):
        bb = ebase + j * CH
        pltpu.sync_copy(row_hbm.at[pl.ds(bb, CH)], rowbuf)
        pltpu.sync_copy(col_hbm.at[pl.ds(bb, CH)], colbuf)
        pltpu.sync_copy(ec_hbm.at[pl.ds(bb, CH)], ecbuf)
        pltpu.sync_copy(h_hbm.at[rowbuf], bufG)

        def group(g, _):
            off = g * 16
            ecv = ecbuf[pl.ds(off, 16)]
            rv = rowbuf[pl.ds(off, 16)]
            for i in range(16):
                e = off + i
                dval = dis_v[pl.ds(rv[i], 16)][0]
                ecs = ecv[i]
                for q in range(D // 16):
                    sl = pl.ds(q * 16, 16)
                    m = jnp.maximum(bufG[e, sl] + ct_v[ecs, sl], 0.0) * dval
                    bufM[e, sl] = m
            return 0
        lax.fori_loop(0, GROUPS, group, 0)

        pltpu.sync_copy(bufM, agg_sp.at[colbuf], add=True)
        return 0
    lax.fori_loop(0, CHUNKS, chunk, 0)
    plsc.subcore_barrier()

    pltpu.sync_copy(agg_sp.at[pl.ds(r0, SL)], aggp_hbm.at[c, pl.ds(r0, SL)])


def _aggregate(h, dis, row, col, ec, ct):
    mesh = plsc.VectorSubcoreMesh(core_axis_name="c", subcore_axis_name="s")
    f = functools.partial(
        pl.kernel,
        mesh=mesh,
        out_type=jax.ShapeDtypeStruct((NC, NPAD, D), jnp.float32),
        scratch_types=[
            pltpu.VMEM_SHARED((NPAD, D), jnp.float32),
            pltpu.VMEM((60, D), jnp.float32),
            pltpu.VMEM((NPAD + 16,), jnp.float32),
            pltpu.VMEM((CH,), jnp.int32),
            pltpu.VMEM((CH,), jnp.int32),
            pltpu.VMEM((CH,), jnp.int32),
            pltpu.VMEM((CH, D), jnp.float32),
            pltpu.VMEM((CH, D), jnp.float32),
        ],
    )(_k3_body)
    return f(h, dis, row, col, ec, ct)


def _p4_body(aggp, dis, selfsc, o):
    o[...] = (aggp[0] + aggp[1]) * dis[...][:, None] + selfsc[...]


def _combine(aggp, dis, selfsc):
    return pl.pallas_call(
        _p4_body,
        out_shape=jax.ShapeDtypeStruct((NPAD, D), jnp.float32),
    )(aggp, dis, selfsc)


def kernel(x, edge_index, edge_attr, W, b, root_emb, bond_emb0, bond_emb1,
           bond_emb2):
    row = edge_index[0]
    col = edge_index[1]
    xp = jnp.pad(x, ((0, NPAD - N), (0, 0)))
    h = _matmul(xp, W, b)
    ec = _edge_codes(edge_attr[:, 0], edge_attr[:, 1], edge_attr[:, 2])
    ct = _bond_table(bond_emb0, bond_emb1, bond_emb2)
    degp = _degree(row)
    dis, selfsc = _norms(h, degp, root_emb)
    aggp = _aggregate(h, dis, row, col, ec, ct)
    out = _combine(aggp, dis, selfsc)
    return out[:N]


# speedup vs baseline: 12.0703x; 1.3064x over previous
import functools

import jax
import jax.numpy as jnp
from jax import lax
from jax.experimental import pallas as pl
from jax.experimental.pallas import tpu as pltpu
from jax.experimental.pallas import tpu_sc as plsc

N = 10000
E = 320000
D = 128
NT = 16
NC = 2
NPAD = 10240
SL = NPAD // NT

CH = 80
PER_TILE = (E // NC) // NT
CHUNKS = PER_TILE // CH
GROUPS = CH // 16


def _mm_body(x_ref, w_ref, b_ref, out_ref):
    y = lax.dot_general(x_ref[...], w_ref[...], (((1,), (1,)), ((), ())),
                        preferred_element_type=jnp.float32)
    out_ref[...] = y + b_ref[...]


def _matmul(xp, W, b):
    return pl.pallas_call(
        _mm_body,
        grid=(16,),
        in_specs=[
            pl.BlockSpec((SL, D), lambda i: (i, 0)),
            pl.BlockSpec((D, D), lambda i: (0, 0)),
            pl.BlockSpec((1, D), lambda i: (0, 0)),
        ],
        out_specs=pl.BlockSpec((SL, D), lambda i: (i, 0)),
        out_shape=jax.ShapeDtypeStruct((NPAD, D), jnp.float32),
    )(xp, W, b.reshape(1, D))


def _ec_body(a0, a1, a2, o):
    o[...] = a0[...] * 12 + a1[...] * 2 + a2[...]


def _edge_codes(a0, a1, a2):
    return pl.pallas_call(
        _ec_body,
        out_shape=jax.ShapeDtypeStruct((E,), jnp.int32),
    )(a0, a1, a2)


def _ct_body(b0, b1, b2, o):
    t = (b0[...][:, None, None, :] + b1[...][None, :, None, :]
         + b2[...][None, None, :, :])
    o[...] = t.reshape(60, D)


def _bond_table(b0, b1, b2):
    return pl.pallas_call(
        _ct_body,
        out_shape=jax.ShapeDtypeStruct((60, D), jnp.float32),
    )(b0, b1, b2)


def _deg_body(row_hbm, degp_hbm, deg_sp, rowbuf, ones_v, zbuf):
    c = lax.axis_index("c")
    s = lax.axis_index("s")

    def zero_blk(i, _):
        zbuf[pl.ds(i * 16, 16)] = jnp.zeros((16,), jnp.float32)
        return 0
    lax.fori_loop(0, SL // 16, zero_blk, 0)
    pltpu.sync_copy(zbuf, deg_sp.at[pl.ds(s * SL, SL)])

    def one_blk(i, _):
        ones_v[pl.ds(i * 16, 16)] = jnp.ones((16,), jnp.float32)
        return 0
    lax.fori_loop(0, GROUPS, one_blk, 0)
    plsc.subcore_barrier()

    base0 = c * (E // NC) + s * PER_TILE

    def chunk(j, _):
        pltpu.sync_copy(row_hbm.at[pl.ds(base0 + j * CH, CH)], rowbuf)
        pltpu.sync_copy(ones_v, deg_sp.at[rowbuf], add=True)
        return 0
    lax.fori_loop(0, CHUNKS, chunk, 0)
    plsc.subcore_barrier()

    @pl.when(s == 0)
    def _():
        pltpu.sync_copy(deg_sp, degp_hbm.at[c])


def _degree(row):
    mesh = plsc.VectorSubcoreMesh(core_axis_name="c", subcore_axis_name="s")
    f = functools.partial(
        pl.kernel,
        mesh=mesh,
        out_type=jax.ShapeDtypeStruct((NC, NPAD), jnp.float32),
        scratch_types=[
            pltpu.VMEM_SHARED((NPAD,), jnp.float32),
            pltpu.VMEM((CH,), jnp.int32),
            pltpu.VMEM((CH,), jnp.float32),
            pltpu.VMEM((SL,), jnp.float32),
        ],
    )(_deg_body)
    return f(row)


def _p3_body(h, degp, root, dis_o, self_o):
    deg = degp[0] + degp[1] + 1.0
    dis_o[...] = lax.rsqrt(deg)
    self_o[...] = jnp.maximum(h[...] + root[...], 0.0) * (1.0 / deg)[:, None]


def _norms(h, degp, root):
    return pl.pallas_call(
        _p3_body,
        out_shape=[
            jax.ShapeDtypeStruct((NPAD,), jnp.float32),
            jax.ShapeDtypeStruct((NPAD, D), jnp.float32),
        ],
    )(h, degp, root)


NAGG = N


def _k3_body(h_hbm, dis_hbm, row_hbm, col_hbm, ec_hbm, ct_hbm, aggp_hbm,
             agg_sp, ct_v, dis_v, rowbuf0, colbuf0, ecbuf0, rowbuf1, colbuf1,
             ecbuf1, bufG0, bufG1, bufM, semG0, semG1, semI0, semI1):
    c = lax.axis_index("c")
    s = lax.axis_index("s")
    r0 = s * SL
    rowbuf = [rowbuf0, rowbuf1]
    colbuf = [colbuf0, colbuf1]
    ecbuf = [ecbuf0, ecbuf1]
    bufG = [bufG0, bufG1]
    semG = [semG0, semG1]
    semI = [semI0, semI1]

    def zrow(i, _):
        for q in range(D // 16):
            bufM[i, pl.ds(q * 16, 16)] = jnp.zeros((16,), jnp.float32)
        return 0
    lax.fori_loop(0, CH, zrow, 0)

    @pl.when(s < NT - 1)
    def _():
        def zcp(i, _):
            pltpu.sync_copy(bufM, agg_sp.at[pl.ds(r0 + i * CH, CH)])
            return 0
        lax.fori_loop(0, SL // CH, zcp, 0)

    @pl.when(s == NT - 1)
    def _():
        def zcp(i, _):
            pltpu.sync_copy(bufM, agg_sp.at[pl.ds(r0 + i * CH, CH)])
            return 0
        lax.fori_loop(0, (NAGG - (NT - 1) * SL) // CH, zcp, 0)

    pltpu.sync_copy(ct_hbm, ct_v)
    pltpu.sync_copy(dis_hbm, dis_v.at[pl.ds(0, NPAD)])
    plsc.subcore_barrier()

    ebase = c * (E // NC) + s * PER_TILE

    def load_idx_sync(j, b):
        bb = ebase + j * CH
        pltpu.sync_copy(row_hbm.at[pl.ds(bb, CH)], rowbuf[b])
        pltpu.sync_copy(col_hbm.at[pl.ds(bb, CH)], colbuf[b])
        pltpu.sync_copy(ec_hbm.at[pl.ds(bb, CH)], ecbuf[b])

    def issue_idx(j, b):
        bb = ebase + j * CH
        pltpu.async_copy(row_hbm.at[pl.ds(bb, CH)], rowbuf[b], semI[b])
        pltpu.async_copy(col_hbm.at[pl.ds(bb, CH)], colbuf[b], semI[b])
        pltpu.async_copy(ec_hbm.at[pl.ds(bb, CH)], ecbuf[b], semI[b])

    def wait_idx(j, b):
        bb = ebase + j * CH
        pltpu.make_async_copy(row_hbm.at[pl.ds(bb, CH)], rowbuf[b], semI[b]).wait()
        pltpu.make_async_copy(col_hbm.at[pl.ds(bb, CH)], colbuf[b], semI[b]).wait()
        pltpu.make_async_copy(ec_hbm.at[pl.ds(bb, CH)], ecbuf[b], semI[b]).wait()

    def step(j, b, issue_gather, prefetch_idx):
        pltpu.make_async_copy(h_hbm.at[rowbuf[b]], bufG[b], semG[b]).wait()
        if issue_gather:
            wait_idx(j + 1, b ^ 1)
            pltpu.async_copy(h_hbm.at[rowbuf[b ^ 1]], bufG[b ^ 1], semG[b ^ 1])

        def group(g, _):
            off = g * 16
            ecv = ecbuf[b][pl.ds(off, 16)]
            rv = rowbuf[b][pl.ds(off, 16)]
            for i in range(16):
                e = off + i
                dval = dis_v[pl.ds(rv[i], 16)][0]
                ecs = ecv[i]
                for q in range(D // 16):
                    sl = pl.ds(q * 16, 16)
                    m = jnp.maximum(bufG[b][e, sl] + ct_v[ecs, sl], 0.0) * dval
                    bufM[e, sl] = m
            return 0
        lax.fori_loop(0, GROUPS, group, 0)

        pltpu.sync_copy(bufM, agg_sp.at[colbuf[b]], add=True)
        if prefetch_idx:
            @pl.when(j + 2 < CHUNKS)
            def _():
                issue_idx(j + 2, b)

    load_idx_sync(0, 0)
    pltpu.async_copy(h_hbm.at[rowbuf[0]], bufG[0], semG[0])
    issue_idx(1, 1)

    def pair(p, _):
        step(2 * p, 0, True, True)
        step(2 * p + 1, 1, True, True)
        return 0
    lax.fori_loop(0, (CHUNKS - 1) // 2, pair, 0)
    step(CHUNKS - 1, 0, False, False)

    plsc.subcore_barrier()

    @pl.when(s < NT - 1)
    def _():
        pltpu.sync_copy(agg_sp.at[pl.ds(r0, SL)], aggp_hbm.at[c, pl.ds(r0, SL)])

    @pl.when(s == NT - 1)
    def _():
        tail = NAGG - (NT - 1) * SL
        pltpu.sync_copy(agg_sp.at[pl.ds(r0, tail)],
                        aggp_hbm.at[c, pl.ds(r0, tail)])


def _aggregate(h, dis, row, col, ec, ct):
    mesh = plsc.VectorSubcoreMesh(core_axis_name="c", subcore_axis_name="s")
    f = functools.partial(
        pl.kernel,
        mesh=mesh,
        out_type=jax.ShapeDtypeStruct((NC, NAGG, D), jnp.float32),
        scratch_types=[
            pltpu.VMEM_SHARED((NAGG, D), jnp.float32),
            pltpu.VMEM((60, D), jnp.float32),
            pltpu.VMEM((NPAD + 16,), jnp.float32),
            pltpu.VMEM((CH,), jnp.int32),
            pltpu.VMEM((CH,), jnp.int32),
            pltpu.VMEM((CH,), jnp.int32),
            pltpu.VMEM((CH,), jnp.int32),
            pltpu.VMEM((CH,), jnp.int32),
            pltpu.VMEM((CH,), jnp.int32),
            pltpu.VMEM((CH, D), jnp.float32),
            pltpu.VMEM((CH, D), jnp.float32),
            pltpu.VMEM((CH, D), jnp.float32),
            pltpu.SemaphoreType.DMA,
            pltpu.SemaphoreType.DMA,
            pltpu.SemaphoreType.DMA,
            pltpu.SemaphoreType.DMA,
        ],
    )(_k3_body)
    return f(h, dis, row, col, ec, ct)


def _p4_body(aggp, dis, selfsc, o):
    o[...] = (aggp[0] + aggp[1]) * dis[...][:, None] + selfsc[...]


def _combine(aggp, dis, selfsc):
    return pl.pallas_call(
        _p4_body,
        out_shape=jax.ShapeDtypeStruct((N, D), jnp.float32),
    )(aggp, dis, selfsc)


def kernel(x, edge_index, edge_attr, W, b, root_emb, bond_emb0, bond_emb1,
           bond_emb2):
    row = edge_index[0]
    col = edge_index[1]
    xp = jnp.pad(x, ((0, NPAD - N), (0, 0)))
    h = _matmul(xp, W, b)
    ec = _edge_codes(edge_attr[:, 0], edge_attr[:, 1], edge_attr[:, 2])
    ct = _bond_table(bond_emb0, bond_emb1, bond_emb2)
    degp = _degree(row)
    dis, selfsc = _norms(h, degp, root_emb)
    aggp = _aggregate(h, dis, row, col, ec, ct)
    out = _combine(aggp, dis[:N], selfsc[:N])
    return out


# speedup vs baseline: 12.6679x; 1.0495x over previous
import functools

import jax
import jax.numpy as jnp
from jax import lax
from jax.experimental import pallas as pl
from jax.experimental.pallas import tpu as pltpu
from jax.experimental.pallas import tpu_sc as plsc

N = 10000
E = 320000
D = 128
NT = 16
NC = 2
NPAD = 10240
SL = NPAD // NT

CH = 80
PER_TILE = (E // NC) // NT
CHUNKS = PER_TILE // CH
GROUPS = CH // 16


def _mm_body(x_ref, w_ref, b_ref, out_ref):
    y = lax.dot_general(x_ref[...], w_ref[...], (((1,), (1,)), ((), ())),
                        preferred_element_type=jnp.float32)
    out_ref[...] = y + b_ref[...]


def _matmul(xp, W, b):
    return pl.pallas_call(
        _mm_body,
        grid=(16,),
        in_specs=[
            pl.BlockSpec((SL, D), lambda i: (i, 0)),
            pl.BlockSpec((D, D), lambda i: (0, 0)),
            pl.BlockSpec((1, D), lambda i: (0, 0)),
        ],
        out_specs=pl.BlockSpec((SL, D), lambda i: (i, 0)),
        out_shape=jax.ShapeDtypeStruct((NPAD, D), jnp.float32),
    )(xp, W, b.reshape(1, D))


def _ec_body(a0, a1, a2, o):
    o[...] = a0[...] * 12 + a1[...] * 2 + a2[...]


def _edge_codes(a0, a1, a2):
    return pl.pallas_call(
        _ec_body,
        out_shape=jax.ShapeDtypeStruct((E,), jnp.int32),
    )(a0, a1, a2)


def _ct_body(b0, b1, b2, o):
    t = (b0[...][:, None, None, :] + b1[...][None, :, None, :]
         + b2[...][None, None, :, :])
    o[...] = t.reshape(60, D)


def _bond_table(b0, b1, b2):
    return pl.pallas_call(
        _ct_body,
        out_shape=jax.ShapeDtypeStruct((60, D), jnp.float32),
    )(b0, b1, b2)


def _deg_body(row_hbm, degp_hbm, deg_sp, rowbuf, ones_v, zbuf):
    c = lax.axis_index("c")
    s = lax.axis_index("s")

    def zero_blk(i, _):
        zbuf[pl.ds(i * 16, 16)] = jnp.zeros((16,), jnp.float32)
        return 0
    lax.fori_loop(0, SL // 16, zero_blk, 0)
    pltpu.sync_copy(zbuf, deg_sp.at[pl.ds(s * SL, SL)])

    def one_blk(i, _):
        ones_v[pl.ds(i * 16, 16)] = jnp.ones((16,), jnp.float32)
        return 0
    lax.fori_loop(0, GROUPS, one_blk, 0)
    plsc.subcore_barrier()

    base0 = c * (E // NC) + s * PER_TILE

    def chunk(j, _):
        pltpu.sync_copy(row_hbm.at[pl.ds(base0 + j * CH, CH)], rowbuf)
        pltpu.sync_copy(ones_v, deg_sp.at[rowbuf], add=True)
        return 0
    lax.fori_loop(0, CHUNKS, chunk, 0)
    plsc.subcore_barrier()

    @pl.when(s == 0)
    def _():
        pltpu.sync_copy(deg_sp, degp_hbm.at[c])


def _degree(row):
    mesh = plsc.VectorSubcoreMesh(core_axis_name="c", subcore_axis_name="s")
    f = functools.partial(
        pl.kernel,
        mesh=mesh,
        out_type=jax.ShapeDtypeStruct((NC, NPAD), jnp.float32),
        scratch_types=[
            pltpu.VMEM_SHARED((NPAD,), jnp.float32),
            pltpu.VMEM((CH,), jnp.int32),
            pltpu.VMEM((CH,), jnp.float32),
            pltpu.VMEM((SL,), jnp.float32),
        ],
    )(_deg_body)
    return f(row)


def _p3_body(h, degp, root, dis_o, self_o):
    deg = degp[0] + degp[1] + 1.0
    dis_o[...] = lax.rsqrt(deg)
    self_o[...] = jnp.maximum(h[...] + root[...], 0.0) * (1.0 / deg)[:, None]


def _norms(h, degp, root):
    return pl.pallas_call(
        _p3_body,
        out_shape=[
            jax.ShapeDtypeStruct((NPAD,), jnp.float32),
            jax.ShapeDtypeStruct((NPAD, D), jnp.float32),
        ],
    )(h, degp, root)


NAGG = N


def _k3_body(h_hbm, dis_hbm, row_hbm, col_hbm, ec_hbm, ct_hbm, aggp_hbm,
             agg_sp, ct_v, dis_v,
             rowbuf0, colbuf0, ecbuf0, rowbuf1, colbuf1, ecbuf1,
             rowbuf2, colbuf2, ecbuf2, rowbuf3, colbuf3, ecbuf3,
             bufG0, bufG1, semG0, semG1, semS0, semS1,
             semI0, semI1, semI2, semI3):
    c = lax.axis_index("c")
    s = lax.axis_index("s")
    r0 = s * SL
    rowbuf = [rowbuf0, rowbuf1, rowbuf2, rowbuf3]
    colbuf = [colbuf0, colbuf1, colbuf2, colbuf3]
    ecbuf = [ecbuf0, ecbuf1, ecbuf2, ecbuf3]
    bufG = [bufG0, bufG1]
    semG = [semG0, semG1]
    semS = [semS0, semS1]
    semI = [semI0, semI1, semI2, semI3]

    def zrow(i, _):
        for q in range(D // 16):
            bufG0[i, pl.ds(q * 16, 16)] = jnp.zeros((16,), jnp.float32)
        return 0
    lax.fori_loop(0, CH, zrow, 0)

    @pl.when(s < NT - 1)
    def _():
        def zcp(i, _):
            pltpu.sync_copy(bufG0, agg_sp.at[pl.ds(r0 + i * CH, CH)])
            return 0
        lax.fori_loop(0, SL // CH, zcp, 0)

    @pl.when(s == NT - 1)
    def _():
        def zcp(i, _):
            pltpu.sync_copy(bufG0, agg_sp.at[pl.ds(r0 + i * CH, CH)])
            return 0
        lax.fori_loop(0, (NAGG - (NT - 1) * SL) // CH, zcp, 0)

    pltpu.sync_copy(ct_hbm, ct_v)
    pltpu.sync_copy(dis_hbm, dis_v.at[pl.ds(0, NPAD)])
    plsc.subcore_barrier()

    ebase = c * (E // NC) + s * PER_TILE

    def issue_idx(j, t):
        bb = ebase + j * CH
        pltpu.async_copy(row_hbm.at[pl.ds(bb, CH)], rowbuf[t], semI[t])
        pltpu.async_copy(col_hbm.at[pl.ds(bb, CH)], colbuf[t], semI[t])
        pltpu.async_copy(ec_hbm.at[pl.ds(bb, CH)], ecbuf[t], semI[t])

    def wait_idx(j, t):
        bb = ebase + j * CH
        pltpu.make_async_copy(row_hbm.at[pl.ds(bb, CH)], rowbuf[t], semI[t]).wait()
        pltpu.make_async_copy(col_hbm.at[pl.ds(bb, CH)], colbuf[t], semI[t]).wait()
        pltpu.make_async_copy(ec_hbm.at[pl.ds(bb, CH)], ecbuf[t], semI[t]).wait()

    def step(j, b, t, first, issue_gather, prefetch_idx, sync_scatter):
        pltpu.make_async_copy(h_hbm.at[rowbuf[t]], bufG[b], semG[b]).wait()
        if not first:
            pltpu.make_async_copy(
                bufG[b ^ 1], agg_sp.at[colbuf[(t + 3) % 4]], semS[b ^ 1]).wait()
        if issue_gather:
            wait_idx(j + 1, (t + 1) % 4)
            pltpu.async_copy(h_hbm.at[rowbuf[(t + 1) % 4]], bufG[b ^ 1],
                             semG[b ^ 1])

        def group(g, _):
            off = g * 16
            ecv = ecbuf[t][pl.ds(off, 16)]
            rv = rowbuf[t][pl.ds(off, 16)]
            for i in range(16):
                e = off + i
                dval = dis_v[pl.ds(rv[i], 16)][0]
                ecs = ecv[i]
                for q in range(D // 16):
                    sl = pl.ds(q * 16, 16)
                    m = jnp.maximum(bufG[b][e, sl] + ct_v[ecs, sl], 0.0) * dval
                    bufG[b][e, sl] = m
            return 0
        lax.fori_loop(0, GROUPS, group, 0)

        if sync_scatter:
            pltpu.sync_copy(bufG[b], agg_sp.at[colbuf[t]], add=True)
        else:
            pltpu.async_copy(bufG[b], agg_sp.at[colbuf[t]], semS[b], add=True)
        if prefetch_idx:
            @pl.when(j + 2 < CHUNKS)
            def _():
                issue_idx(j + 2, (t + 2) % 4)

    bb0 = ebase
    pltpu.sync_copy(row_hbm.at[pl.ds(bb0, CH)], rowbuf[0])
    pltpu.sync_copy(col_hbm.at[pl.ds(bb0, CH)], colbuf[0])
    pltpu.sync_copy(ec_hbm.at[pl.ds(bb0, CH)], ecbuf[0])
    pltpu.async_copy(h_hbm.at[rowbuf[0]], bufG[0], semG[0])
    issue_idx(1, 1)

    step(0, 0, 0, True, True, True, False)
    step(1, 1, 1, False, True, True, False)
    step(2, 0, 2, False, True, True, False)
    step(3, 1, 3, False, True, True, False)

    def quad(p, _):
        j = p * 4
        step(j, 0, 0, False, True, True, False)
        step(j + 1, 1, 1, False, True, True, False)
        step(j + 2, 0, 2, False, True, True, False)
        step(j + 3, 1, 3, False, True, True, False)
        return 0
    lax.fori_loop(1, (CHUNKS - 1) // 4, quad, 0)
    step(CHUNKS - 1, 0, 0, False, False, False, True)

    plsc.subcore_barrier()

    @pl.when(s < NT - 1)
    def _():
        pltpu.sync_copy(agg_sp.at[pl.ds(r0, SL)], aggp_hbm.at[c, pl.ds(r0, SL)])

    @pl.when(s == NT - 1)
    def _():
        tail = NAGG - (NT - 1) * SL
        pltpu.sync_copy(agg_sp.at[pl.ds(r0, tail)],
                        aggp_hbm.at[c, pl.ds(r0, tail)])


def _aggregate(h, dis, row, col, ec, ct):
    mesh = plsc.VectorSubcoreMesh(core_axis_name="c", subcore_axis_name="s")
    f = functools.partial(
        pl.kernel,
        mesh=mesh,
        out_type=jax.ShapeDtypeStruct((NC, NAGG, D), jnp.float32),
        scratch_types=[
            pltpu.VMEM_SHARED((NAGG, D), jnp.float32),
            pltpu.VMEM((60, D), jnp.float32),
            pltpu.VMEM((NPAD + 16,), jnp.float32),
        ] + [pltpu.VMEM((CH,), jnp.int32)] * 12 + [
            pltpu.VMEM((CH, D), jnp.float32),
            pltpu.VMEM((CH, D), jnp.float32),
        ] + [pltpu.SemaphoreType.DMA] * 8,
    )(_k3_body)
    return f(h, dis, row, col, ec, ct)


def _p4_body(aggp, dis, selfsc, o):
    o[...] = (aggp[0] + aggp[1]) * dis[...][:, None] + selfsc[...]


def _combine(aggp, dis, selfsc):
    return pl.pallas_call(
        _p4_body,
        out_shape=jax.ShapeDtypeStruct((N, D), jnp.float32),
    )(aggp, dis, selfsc)


def kernel(x, edge_index, edge_attr, W, b, root_emb, bond_emb0, bond_emb1,
           bond_emb2):
    row = edge_index[0]
    col = edge_index[1]
    xp = jnp.pad(x, ((0, NPAD - N), (0, 0)))
    h = _matmul(xp, W, b)
    ec = _edge_codes(edge_attr[:, 0], edge_attr[:, 1], edge_attr[:, 2])
    ct = _bond_table(bond_emb0, bond_emb1, bond_emb2)
    degp = _degree(row)
    dis, selfsc = _norms(h, degp, root_emb)
    aggp = _aggregate(h, dis, row, col, ec, ct)
    out = _combine(aggp, dis[:N], selfsc[:N])
    return out
